# baseline (speedup 1.0000x reference)
import functools

import jax
import jax.numpy as jnp
from jax import lax
from jax.experimental import pallas as pl
from jax.experimental.pallas import tpu as pltpu
from jax.experimental.pallas import tpu_sc as plsc

NUM_CAMERAS = 100000
BATCH = 16384
D = 3
DP = 4
L = 16
NC, NS = 2, 16
NW = NC * NS
BPW = BATCH // NW
TW = BPW * D
CHUNK = 128
NCHUNK = TW // CHUNK
WORDS = BPW * 2 * D

_mesh = plsc.VectorSubcoreMesh(core_axis_name="c", subcore_axis_name="s")


@functools.partial(
    pl.kernel,
    mesh=_mesh,
    compiler_params=pltpu.CompilerParams(use_tc_tiling_on_sc=True,
                                         needs_layout_passes=False),
    out_type=jax.ShapeDtypeStruct((BATCH * 2 * D,), jnp.float32),
    scratch_types=[
        pltpu.VMEM((BPW,), jnp.int32),
        pltpu.VMEM((TW,), jnp.int32),
        pltpu.VMEM((2 * TW,), jnp.float32),
        pltpu.VMEM((WORDS,), jnp.float32),
        pltpu.SemaphoreType.DMA,
    ],
)
def _gather_add(init_hbm, idx_hbm, lin_hbm, ang_hbm, out_hbm,
                idx_v, gidx_v, tab_v, acc_v, sem):
    wid = lax.axis_index("s") * NC + lax.axis_index("c")

    pltpu.sync_copy(idx_hbm.at[pl.ds(wid * BPW, BPW)], idx_v)

    def expand(i, carry):
        q = lax.iota(jnp.int32, L) + i * L
        iv = plsc.load_gather(idx_v, [lax.div(q, D)])
        gidx_v[pl.ds(i * L, L)] = iv * DP + lax.rem(q, D)
        return carry

    lax.fori_loop(0, TW // L, expand, 0)

    copies = []
    for k in range(NCHUNK):
        g = gidx_v.at[pl.ds(k * CHUNK, CHUNK)]
        copies.append(pltpu.async_copy(
            lin_hbm.at[g], tab_v.at[pl.ds(k * CHUNK, CHUNK)], sem))
        copies.append(pltpu.async_copy(
            ang_hbm.at[g], tab_v.at[pl.ds(TW + k * CHUNK, CHUNK)], sem))

    pltpu.sync_copy(init_hbm.at[pl.ds(wid * WORDS, WORDS)], acc_v)
    for c in copies:
        c.wait()

    def body(i, carry):
        q = lax.iota(jnp.int32, L) + i * L
        src = (lax.div(q, 6) * D + lax.rem(q, D)
               + TW * lax.div(lax.rem(q, 6), D))
        adj = plsc.load_gather(tab_v, [src])
        acc_v[pl.ds(i * L, L)] = acc_v[pl.ds(i * L, L)] + adj
        return carry

    lax.fori_loop(0, WORDS // L, body, 0)

    pltpu.sync_copy(acc_v, out_hbm.at[pl.ds(wid * WORDS, WORDS)])


def kernel(init_velocities, camera_idx, linear_velocity_adjustment,
           angular_velocity_adjustment):
    pad = ((0, 0), (0, DP - D))
    lin4 = jnp.pad(linear_velocity_adjustment, pad).reshape(-1)
    ang4 = jnp.pad(angular_velocity_adjustment, pad).reshape(-1)
    out = _gather_add(init_velocities.reshape(-1),
                      camera_idx.astype(jnp.int32), lin4, ang4)
    return out.reshape(BATCH, 2 * D)


# speedup vs baseline: 1.3981x; 1.3981x over previous
import functools

import jax
import jax.numpy as jnp
from jax import lax
from jax.experimental import pallas as pl
from jax.experimental.pallas import tpu as pltpu
from jax.experimental.pallas import tpu_sc as plsc

NUM_CAMERAS = 100000
BATCH = 16384
D = 3
L = 16
NC, NS = 2, 16
NW = NC * NS
BPW = BATCH // NW
TW = BPW * D
TOFF = NUM_CAMERAS * D
CHUNK = 128
NCHUNK = TW // CHUNK
WORDS = BPW * 2 * D

_mesh = plsc.VectorSubcoreMesh(core_axis_name="c", subcore_axis_name="s")


@functools.partial(
    pl.kernel,
    mesh=_mesh,
    compiler_params=pltpu.CompilerParams(use_tc_tiling_on_sc=False,
                                         needs_layout_passes=False),
    out_type=jax.ShapeDtypeStruct((BATCH * 2 * D,), jnp.float32),
    scratch_types=[
        pltpu.VMEM((BPW,), jnp.int32),
        pltpu.VMEM((2 * TW,), jnp.int32),
        pltpu.VMEM((2 * TW,), jnp.float32),
        pltpu.VMEM((WORDS,), jnp.float32),
        pltpu.SemaphoreType.DMA,
    ],
)
def _gather_add(init_hbm, idx_hbm, tab_hbm, out_hbm,
                idx_v, gidx_v, tab_v, acc_v, sem):
    wid = lax.axis_index("s") * NC + lax.axis_index("c")

    pltpu.sync_copy(idx_hbm.at[pl.ds(wid * BPW, BPW)], idx_v)

    def expand(i, carry):
        q = lax.iota(jnp.int32, L) + i * L
        iv = plsc.load_gather(idx_v, [lax.div(q, D)])
        gidx_v[pl.ds(i * L, L)] = iv * D + lax.rem(q, D)
        gidx_v[pl.ds(TW + i * L, L)] = iv * D + lax.rem(q, D) + TOFF
        return carry

    lax.fori_loop(0, TW // L, expand, 0)

    copies = []
    for k in range(2 * NCHUNK):
        copies.append(pltpu.async_copy(
            tab_hbm.at[gidx_v.at[pl.ds(k * CHUNK, CHUNK)]],
            tab_v.at[pl.ds(k * CHUNK, CHUNK)], sem))

    pltpu.sync_copy(init_hbm.at[pl.ds(wid * WORDS, WORDS)], acc_v)
    for c in copies:
        c.wait()

    def body(i, carry):
        q = lax.iota(jnp.int32, L) + i * L
        src = (lax.div(q, 6) * D + lax.rem(q, D)
               + TW * lax.div(lax.rem(q, 6), D))
        adj = plsc.load_gather(tab_v, [src])
        acc_v[pl.ds(i * L, L)] = acc_v[pl.ds(i * L, L)] + adj
        return carry

    lax.fori_loop(0, WORDS // L, body, 0)

    pltpu.sync_copy(acc_v, out_hbm.at[pl.ds(wid * WORDS, WORDS)])


def kernel(init_velocities, camera_idx, linear_velocity_adjustment,
           angular_velocity_adjustment):
    tab = jnp.concatenate([linear_velocity_adjustment,
                           angular_velocity_adjustment], axis=0).reshape(-1)
    out = _gather_add(init_velocities.reshape(-1),
                      camera_idx.astype(jnp.int32), tab)
    return out.reshape(BATCH, 2 * D)


# speedup vs baseline: 1.4059x; 1.0056x over previous
import functools

import jax
import jax.numpy as jnp
from jax import lax
from jax.experimental import pallas as pl
from jax.experimental.pallas import tpu as pltpu
from jax.experimental.pallas import tpu_sc as plsc

NUM_CAMERAS = 100000
BATCH = 16384
D = 3
L = 16
NC, NS = 2, 16
NW = NC * NS
BPW = BATCH // NW
TW = BPW * D
CHUNK = 128
NCHUNK = TW // CHUNK
WORDS = BPW * 2 * D

_mesh = plsc.VectorSubcoreMesh(core_axis_name="c", subcore_axis_name="s")


@functools.partial(
    pl.kernel,
    mesh=_mesh,
    compiler_params=pltpu.CompilerParams(use_tc_tiling_on_sc=False,
                                         needs_layout_passes=False),
    out_type=jax.ShapeDtypeStruct((BATCH * 2 * D,), jnp.float32),
    scratch_types=[
        pltpu.VMEM((BPW,), jnp.int32),
        pltpu.VMEM((TW,), jnp.int32),
        pltpu.VMEM((2 * TW,), jnp.float32),
        pltpu.VMEM((WORDS,), jnp.float32),
        pltpu.SemaphoreType.DMA,
    ],
)
def _gather_add(init_hbm, idx_hbm, lin_hbm, ang_hbm, out_hbm,
                idx_v, gidx_v, tab_v, acc_v, sem):
    wid = lax.axis_index("s") * NC + lax.axis_index("c")

    pltpu.sync_copy(idx_hbm.at[pl.ds(wid * BPW, BPW)], idx_v)

    def expand(i, carry):
        q = lax.iota(jnp.int32, L) + i * L
        iv = plsc.load_gather(idx_v, [lax.div(q, D)])
        gidx_v[pl.ds(i * L, L)] = iv * D + lax.rem(q, D)
        return carry

    lax.fori_loop(0, TW // L, expand, 0)

    copies = []
    for k in range(NCHUNK):
        g = gidx_v.at[pl.ds(k * CHUNK, CHUNK)]
        copies.append(pltpu.async_copy(
            lin_hbm.at[g], tab_v.at[pl.ds(k * CHUNK, CHUNK)], sem))
        copies.append(pltpu.async_copy(
            ang_hbm.at[g], tab_v.at[pl.ds(TW + k * CHUNK, CHUNK)],
            sem))

    pltpu.sync_copy(init_hbm.at[pl.ds(wid * WORDS, WORDS)], acc_v)
    for c in copies:
        c.wait()

    def body(i, carry):
        q = lax.iota(jnp.int32, L) + i * L
        src = (lax.div(q, 6) * D + lax.rem(q, D)
               + TW * lax.div(lax.rem(q, 6), D))
        adj = plsc.load_gather(tab_v, [src])
        acc_v[pl.ds(i * L, L)] = acc_v[pl.ds(i * L, L)] + adj
        return carry

    lax.fori_loop(0, WORDS // L, body, 0)

    pltpu.sync_copy(acc_v, out_hbm.at[pl.ds(wid * WORDS, WORDS)])


def kernel(init_velocities, camera_idx, linear_velocity_adjustment,
           angular_velocity_adjustment):
    idx = camera_idx.astype(jnp.int32)
    out = _gather_add(init_velocities.reshape(-1), idx,
                      linear_velocity_adjustment.reshape(-1),
                      angular_velocity_adjustment.reshape(-1))
    return out.reshape(BATCH, 2 * D)
